# Initial kernel scaffold; baseline (speedup 1.0000x reference)
import jax
import jax.numpy as jnp
from jax.experimental import pallas as pl


def kernel(logits, ratio, gumbel, z_indices, mask, mask_num):
    raise NotImplementedError("write your pallas kernel here")


# speedup vs baseline: 4.4817x; 4.4817x over previous
import functools

import jax
import jax.numpy as jnp
from jax.experimental import pallas as pl
from jax.experimental.pallas import tpu as pltpu

_CHOICE_TEMPERATURE = 4.5


def _body(temp_ref, ml_ref, logits_ref, gumbel_ref, mask_ref,
          zpred_ref, maskbc_ref, conf_ref, *, s_blk, s, nj):
    j = pl.program_id(1)
    x = logits_ref[0]
    m = jnp.max(x, axis=-1)
    vidx = jax.lax.broadcasted_iota(jnp.int32, x.shape, 1)
    amax = jnp.min(jnp.where(x == m[:, None], vidx, x.shape[-1]), axis=-1)
    se = jnp.sum(jnp.exp(x - m[:, None]), axis=-1)
    pmax = 1.0 / se
    temp = temp_ref[0]
    g = gumbel_ref[0, 0]
    mk = mask_ref[0, 0]
    conf = jnp.where(mk != 0, pmax + temp * g, jnp.inf)
    zpred_ref[0, 0, :] = amax
    conf_ref[0, pl.ds(j * s_blk, s_blk)] = conf

    @pl.when(j == nj - 1)
    def _rank():
        c = conf_ref[0, :]
        ci = c[:, None]
        cj = c[None, :]
        ii = jax.lax.broadcasted_iota(jnp.int32, (s, s), 0)
        jj = jax.lax.broadcasted_iota(jnp.int32, (s, s), 1)
        less = (cj < ci) | ((cj == ci) & (jj < ii))
        rank = jnp.sum(less.astype(jnp.int32), axis=1)
        maskbc_ref[0, 0, :] = (rank < ml_ref[0]).astype(jnp.int32)


def kernel(logits, ratio, gumbel, z_indices, mask, mask_num):
    del z_indices
    b, s, v = logits.shape
    s_blk = 128
    nj = s // s_blk

    r = ratio[0]
    mask_ratio = jnp.cos(r * jnp.pi / 2.0)
    mask_len = jnp.maximum(jnp.ceil(mask_num * mask_ratio), 1.0).astype(jnp.int32)
    temperature = (_CHOICE_TEMPERATURE * (1.0 - mask_ratio)).astype(jnp.float32)

    gumbel3 = gumbel.reshape(b * nj, 1, s_blk)
    mask3 = mask.astype(jnp.int32).reshape(b * nj, 1, s_blk)

    zpred, maskbc = pl.pallas_call(
        functools.partial(_body, s_blk=s_blk, s=s, nj=nj),
        grid=(b, nj),
        in_specs=[
            pl.BlockSpec(memory_space=pltpu.SMEM),
            pl.BlockSpec(memory_space=pltpu.SMEM),
            pl.BlockSpec((1, s_blk, v), lambda bi, ji: (bi, ji, 0)),
            pl.BlockSpec((1, 1, s_blk), lambda bi, ji: (bi * nj + ji, 0, 0)),
            pl.BlockSpec((1, 1, s_blk), lambda bi, ji: (bi * nj + ji, 0, 0)),
        ],
        out_specs=[
            pl.BlockSpec((1, 1, s_blk), lambda bi, ji: (bi * nj + ji, 0, 0)),
            pl.BlockSpec((1, 1, s), lambda bi, ji: (bi, 0, 0)),
        ],
        out_shape=[
            jax.ShapeDtypeStruct((b * nj, 1, s_blk), jnp.int32),
            jax.ShapeDtypeStruct((b, 1, s), jnp.int32),
        ],
        scratch_shapes=[pltpu.VMEM((1, s), jnp.float32)],
    )(temperature.reshape(1), mask_len.reshape(1), logits, gumbel3, mask3)

    return zpred.reshape(b, s), maskbc.reshape(b, s).astype(jnp.bool_)
